# baseline (speedup 1.0000x reference)
import dataclasses

import jax
import jax.numpy as jnp
from jax import lax
from jax.experimental import pallas as pl
from jax.experimental.pallas import tpu as pltpu
from jax.experimental.pallas import tpu_sc as plsc

N = 10000
D = 128
DZ = 32
NC, NS = 2, 16
NW = NC * NS
K = 128
N_PAD = 10112
STRIPE = N_PAD // NS


def _sc_mesh():
    return plsc.VectorSubcoreMesh(
        core_axis_name="c", subcore_axis_name="s", num_cores=NC, num_subcores=NS
    )


def _edge_scatter(hn, srcs, dsts, zeros_d):
    n_chunks = srcs.shape[1]
    half = n_chunks // 2

    scratch = [
        pltpu.VMEM((half, K), jnp.int32),
        pltpu.VMEM((half, K), jnp.int32),
        pltpu.VMEM((K, D), jnp.float32),
        pltpu.VMEM((K, D), jnp.float32),
        pltpu.VMEM_SHARED((N_PAD, D), jnp.float32),
        pltpu.SemaphoreType.DMA,
        pltpu.SemaphoreType.DMA,
        pltpu.SemaphoreType.DMA,
        pltpu.SemaphoreType.DMA,
        pltpu.SemaphoreType.DMA,
    ]

    def body(hn_hbm, src_hbm, dst_hbm, zd_hbm, agg_out,
             src_v, dst_v, rows_a, rows_b, agg_sh,
             sem_a, sem_b, sem_z, sem_sa, sem_sb):
        cid = lax.axis_index("c")
        sid = lax.axis_index("s")
        wid = sid * NC + cid
        r0 = sid * STRIPE

        pltpu.async_copy(zd_hbm.at[pl.ds(r0, STRIPE)],
                         agg_sh.at[pl.ds(r0, STRIPE)], sem_z)
        pltpu.sync_copy(src_hbm.at[wid, pl.ds(0, half)], src_v)
        pltpu.sync_copy(dst_hbm.at[wid, pl.ds(0, half)], dst_v)
        pltpu.make_async_copy(zd_hbm.at[pl.ds(r0, STRIPE)],
                              agg_sh.at[pl.ds(r0, STRIPE)], sem_z).wait()
        plsc.subcore_barrier()

        def start(j, buf, sem):
            pltpu.async_copy(hn_hbm.at[src_v.at[j]], buf, sem)

        def wait(buf, sem):
            pltpu.make_async_copy(hn_hbm.at[src_v.at[0]], buf, sem).wait()

        def scat(j, buf, sem):
            pltpu.async_copy(buf, agg_sh.at[dst_v.at[j]], sem, add=True)

        def scat_wait(buf, sem):
            pltpu.make_async_copy(buf, agg_sh.at[dst_v.at[0]], sem).wait()

        for h in range(2):
            if h:
                pltpu.sync_copy(src_hbm.at[wid, pl.ds(half, half)], src_v)
                pltpu.sync_copy(dst_hbm.at[wid, pl.ds(half, half)], dst_v)
            start(0, rows_a, sem_a)

            @pl.loop(0, half, step=2)
            def _(j):
                wait(rows_a, sem_a)

                @pl.when(j > 0)
                def _():
                    scat_wait(rows_b, sem_sb)
                start(j + 1, rows_b, sem_b)
                scat(j, rows_a, sem_sa)
                wait(rows_b, sem_b)
                scat_wait(rows_a, sem_sa)

                @pl.when(j + 2 < half)
                def _():
                    start(j + 2, rows_a, sem_a)
                scat(j + 1, rows_b, sem_sb)

            scat_wait(rows_b, sem_sb)

        plsc.subcore_barrier()
        pltpu.sync_copy(agg_sh.at[pl.ds(r0, STRIPE)],
                        agg_out.at[cid, pl.ds(r0, STRIPE)])

    run = pl.kernel(body,
                    out_type=jax.ShapeDtypeStruct((NC, N_PAD, D), jnp.float32),
                    mesh=_sc_mesh(), scratch_types=scratch)
    return run(hn, srcs, dsts, zeros_d)


def _edge_degree(dsts, zeros_flat):
    n_chunks = dsts.shape[1]

    scratch = [
        pltpu.VMEM((n_chunks, K), jnp.int32),
        pltpu.VMEM((N_PAD // 128, 128), jnp.float32),
    ]

    def body(dst_hbm, zf_hbm, hist_out, dst_v, hist_v):
        cid = lax.axis_index("c")
        sid = lax.axis_index("s")
        wid = sid * NC + cid
        pltpu.sync_copy(dst_hbm.at[wid], dst_v)
        pltpu.sync_copy(zf_hbm, hist_v)

        @pl.loop(0, n_chunks)
        def _(j):
            for t in range(K // 16):
                idx = dst_v[j, pl.ds(t * 16, 16)]
                cnt, last = plsc.scan_count(idx)
                plsc.addupdate_scatter(
                    hist_v, [idx >> 7, idx & 127],
                    cnt.astype(jnp.float32), mask=last)

        pltpu.sync_copy(hist_v, hist_out.at[wid])

    cp = pltpu.CompilerParams()
    if "needs_layout_passes" in pltpu.CompilerParams.__dataclass_fields__:
        cp = dataclasses.replace(cp, needs_layout_passes=False)
    run = pl.kernel(
        body,
        out_type=jax.ShapeDtypeStruct((NW, N_PAD // 128, 128), jnp.float32),
        mesh=_sc_mesh(), scratch_types=scratch, compiler_params=cp)
    return run(dsts, zeros_flat)


def _tc_in(x, wn, ws, b):
    def body(x_ref, wn_ref, ws_ref, b_ref, hn_ref, s_ref):
        h = jnp.log(x_ref[...] + 1.0)
        hn_ref[...] = jnp.dot(h, wn_ref[...], preferred_element_type=jnp.float32)
        s_ref[...] = jnp.dot(h, ws_ref[...],
                             preferred_element_type=jnp.float32) + b_ref[...]

    return pl.pallas_call(
        body,
        out_shape=(jax.ShapeDtypeStruct((N, D), jnp.float32),
                   jax.ShapeDtypeStruct((N, D), jnp.float32)),
    )(x, wn, ws, b)


def _finish_layer(s_ref, aggp_ref, inv_col):
    agg = aggp_ref[0, :N, :] + aggp_ref[1, :N, :]
    pre = jnp.maximum(s_ref[...] + agg * inv_col, 0.0)
    nrm = jnp.sqrt(jnp.sum(pre * pre, axis=1, keepdims=True))
    return pre / jnp.maximum(nrm, 1e-12)


def _tc_mid(s0, aggp, degp, wn, ws, b):
    def body(s0_ref, aggp_ref, degp_ref, wn_ref, ws_ref, b_ref,
             hn_ref, s_ref, inv_ref):
        degsum = jnp.sum(degp_ref[...], axis=0)
        dt = jnp.transpose(degsum)
        col = jnp.concatenate([dt[:, c:c + 1] for c in range(N_PAD // 128)],
                              axis=0)
        inv = 1.0 / jnp.maximum(col[:N], 1.0)
        inv_ref[...] = inv
        h1 = _finish_layer(s0_ref, aggp_ref, inv)
        hn_ref[...] = jnp.dot(h1, wn_ref[...], preferred_element_type=jnp.float32)
        s_ref[...] = jnp.dot(h1, ws_ref[...],
                             preferred_element_type=jnp.float32) + b_ref[...]

    return pl.pallas_call(
        body,
        out_shape=(jax.ShapeDtypeStruct((N, D), jnp.float32),
                   jax.ShapeDtypeStruct((N, D), jnp.float32),
                   jax.ShapeDtypeStruct((N, 1), jnp.float32)),
    )(s0, aggp, degp, wn, ws, b)


def _tc_out(s1, aggp, inv, wmu, bmu, wvar, bvar):
    def body(s1_ref, aggp_ref, inv_ref, wmu_ref, bmu_ref, wvar_ref, bvar_ref,
             zl_ref, zs_ref):
        h2 = _finish_layer(s1_ref, aggp_ref, inv_ref[...])
        zl_ref[...] = jnp.dot(h2, wmu_ref[...],
                              preferred_element_type=jnp.float32) + bmu_ref[...]
        zs_ref[...] = jnp.exp(jnp.dot(h2, wvar_ref[...],
                                      preferred_element_type=jnp.float32)
                              + bvar_ref[...]) + 1e-6

    return pl.pallas_call(
        body,
        out_shape=(jax.ShapeDtypeStruct((N, DZ), jnp.float32),
                   jax.ShapeDtypeStruct((N, DZ), jnp.float32)),
    )(s1, aggp, inv, wmu, bmu, wvar, bvar)


def kernel(x, edge_index, W_self0, W_neigh0, b0, W_self1, W_neigh1, b1,
           W_mu, b_mu, W_var, b_var):
    n_edges = edge_index.shape[1]
    per_w = -(-n_edges // (NW * 4 * K)) * 4 * K
    pad = per_w * NW - n_edges
    pid = jnp.arange(pad, dtype=jnp.int32)
    srcs = jnp.concatenate([edge_index[0], pid % N]).reshape(NW, per_w // K, K)
    dsts = jnp.concatenate([edge_index[1], N + (pid % (N_PAD - N))]).reshape(
        NW, per_w // K, K)
    zeros_d = jnp.zeros((N_PAD, D), jnp.float32)
    zeros_flat = jnp.zeros((N_PAD // 128, 128), jnp.float32)

    degp = _edge_degree(dsts, zeros_flat)
    hn0, s0 = _tc_in(x, W_neigh0, W_self0, b0.reshape(1, D))
    agg0 = _edge_scatter(hn0, srcs, dsts, zeros_d)
    hn1, s1, inv = _tc_mid(s0, agg0, degp, W_neigh1, W_self1, b1.reshape(1, D))
    agg1 = _edge_scatter(hn1, srcs, dsts, zeros_d)
    return _tc_out(s1, agg1, inv, W_mu, b_mu.reshape(1, DZ),
                   W_var, b_var.reshape(1, DZ))


# speedup vs baseline: 1.1360x; 1.1360x over previous
import dataclasses

import jax
import jax.numpy as jnp
from jax import lax
from jax.experimental import pallas as pl
from jax.experimental.pallas import tpu as pltpu
from jax.experimental.pallas import tpu_sc as plsc

N = 10000
D = 128
DZ = 32
NC, NS = 2, 16
NW = NC * NS
K = 128
N_PAD = 10112
STRIPE = N_PAD // NS


def _sc_mesh():
    return plsc.VectorSubcoreMesh(
        core_axis_name="c", subcore_axis_name="s", num_cores=NC, num_subcores=NS
    )


def _edge_scatter(hn, srcs, dsts, zeros_d):
    n_chunks = srcs.shape[1]
    half = n_chunks // 2

    scratch = [
        pltpu.VMEM((half, K), jnp.int32),
        pltpu.VMEM((half, K), jnp.int32),
        pltpu.VMEM((K, D), jnp.float32),
        pltpu.VMEM((K, D), jnp.float32),
        pltpu.VMEM_SHARED((N_PAD, D), jnp.float32),
        pltpu.SemaphoreType.DMA,
        pltpu.SemaphoreType.DMA,
        pltpu.SemaphoreType.DMA,
    ]

    def body(hn_hbm, src_hbm, dst_hbm, zd_hbm, agg_out,
             src_v, dst_v, rows_a, rows_b, agg_sh, sem_a, sem_b, sem_z):
        cid = lax.axis_index("c")
        sid = lax.axis_index("s")
        wid = sid * NC + cid
        r0 = sid * STRIPE

        pltpu.async_copy(zd_hbm.at[pl.ds(r0, STRIPE)],
                         agg_sh.at[pl.ds(r0, STRIPE)], sem_z)
        pltpu.sync_copy(src_hbm.at[wid, pl.ds(0, half)], src_v)
        pltpu.sync_copy(dst_hbm.at[wid, pl.ds(0, half)], dst_v)
        pltpu.make_async_copy(zd_hbm.at[pl.ds(r0, STRIPE)],
                              agg_sh.at[pl.ds(r0, STRIPE)], sem_z).wait()
        plsc.subcore_barrier()

        def start(j, buf, sem):
            pltpu.async_copy(hn_hbm.at[src_v.at[j]], buf, sem)

        def wait(buf, sem):
            pltpu.make_async_copy(hn_hbm.at[src_v.at[0]], buf, sem).wait()

        def scat(j, buf):
            pltpu.sync_copy(buf, agg_sh.at[dst_v.at[j]], add=True)

        for h in range(2):
            if h:
                pltpu.sync_copy(src_hbm.at[wid, pl.ds(half, half)], src_v)
                pltpu.sync_copy(dst_hbm.at[wid, pl.ds(half, half)], dst_v)
            start(0, rows_a, sem_a)

            @pl.loop(0, half, step=2)
            def _(j):
                start(j + 1, rows_b, sem_b)
                wait(rows_a, sem_a)
                scat(j, rows_a)

                @pl.when(j + 2 < half)
                def _():
                    start(j + 2, rows_a, sem_a)

                wait(rows_b, sem_b)
                scat(j + 1, rows_b)

        plsc.subcore_barrier()
        pltpu.sync_copy(agg_sh.at[pl.ds(r0, STRIPE)],
                        agg_out.at[cid, pl.ds(r0, STRIPE)])

    run = pl.kernel(body,
                    out_type=jax.ShapeDtypeStruct((NC, N_PAD, D), jnp.float32),
                    mesh=_sc_mesh(), scratch_types=scratch)
    return run(hn, srcs, dsts, zeros_d)


def _edge_degree(dsts, zeros_flat):
    n_chunks = dsts.shape[1]

    scratch = [
        pltpu.VMEM((n_chunks, K), jnp.int32),
        pltpu.VMEM((N_PAD // 128, 128), jnp.float32),
    ]

    def body(dst_hbm, zf_hbm, hist_out, dst_v, hist_v):
        cid = lax.axis_index("c")
        sid = lax.axis_index("s")
        wid = sid * NC + cid
        pltpu.sync_copy(dst_hbm.at[wid], dst_v)
        pltpu.sync_copy(zf_hbm, hist_v)

        @pl.loop(0, n_chunks)
        def _(j):
            for t in range(K // 16):
                idx = dst_v[j, pl.ds(t * 16, 16)]
                cnt, last = plsc.scan_count(idx)
                plsc.addupdate_scatter(
                    hist_v, [idx >> 7, idx & 127],
                    cnt.astype(jnp.float32), mask=last)

        pltpu.sync_copy(hist_v, hist_out.at[wid])

    cp = pltpu.CompilerParams()
    if "needs_layout_passes" in pltpu.CompilerParams.__dataclass_fields__:
        cp = dataclasses.replace(cp, needs_layout_passes=False)
    run = pl.kernel(
        body,
        out_type=jax.ShapeDtypeStruct((NW, N_PAD // 128, 128), jnp.float32),
        mesh=_sc_mesh(), scratch_types=scratch, compiler_params=cp)
    return run(dsts, zeros_flat)


def _tc_in(x, wn, ws, b):
    def body(x_ref, wn_ref, ws_ref, b_ref, hn_ref, s_ref):
        h = jnp.log(x_ref[...] + 1.0)
        hn_ref[...] = jnp.dot(h, wn_ref[...], preferred_element_type=jnp.float32)
        s_ref[...] = jnp.dot(h, ws_ref[...],
                             preferred_element_type=jnp.float32) + b_ref[...]

    return pl.pallas_call(
        body,
        out_shape=(jax.ShapeDtypeStruct((N, D), jnp.float32),
                   jax.ShapeDtypeStruct((N, D), jnp.float32)),
    )(x, wn, ws, b)


def _finish_layer(s_ref, aggp_ref, inv_col):
    agg = aggp_ref[0, :N, :] + aggp_ref[1, :N, :]
    pre = jnp.maximum(s_ref[...] + agg * inv_col, 0.0)
    nrm = jnp.sqrt(jnp.sum(pre * pre, axis=1, keepdims=True))
    return pre / jnp.maximum(nrm, 1e-12)


def _tc_mid(s0, aggp, degp, wn, ws, b):
    def body(s0_ref, aggp_ref, degp_ref, wn_ref, ws_ref, b_ref,
             hn_ref, s_ref, inv_ref):
        degsum = jnp.sum(degp_ref[...], axis=0)
        dt = jnp.transpose(degsum)
        col = jnp.concatenate([dt[:, c:c + 1] for c in range(N_PAD // 128)],
                              axis=0)
        inv = 1.0 / jnp.maximum(col[:N], 1.0)
        inv_ref[...] = inv
        h1 = _finish_layer(s0_ref, aggp_ref, inv)
        hn_ref[...] = jnp.dot(h1, wn_ref[...], preferred_element_type=jnp.float32)
        s_ref[...] = jnp.dot(h1, ws_ref[...],
                             preferred_element_type=jnp.float32) + b_ref[...]

    return pl.pallas_call(
        body,
        out_shape=(jax.ShapeDtypeStruct((N, D), jnp.float32),
                   jax.ShapeDtypeStruct((N, D), jnp.float32),
                   jax.ShapeDtypeStruct((N, 1), jnp.float32)),
    )(s0, aggp, degp, wn, ws, b)


def _tc_out(s1, aggp, inv, wmu, bmu, wvar, bvar):
    def body(s1_ref, aggp_ref, inv_ref, wmu_ref, bmu_ref, wvar_ref, bvar_ref,
             zl_ref, zs_ref):
        h2 = _finish_layer(s1_ref, aggp_ref, inv_ref[...])
        zl_ref[...] = jnp.dot(h2, wmu_ref[...],
                              preferred_element_type=jnp.float32) + bmu_ref[...]
        zs_ref[...] = jnp.exp(jnp.dot(h2, wvar_ref[...],
                                      preferred_element_type=jnp.float32)
                              + bvar_ref[...]) + 1e-6

    return pl.pallas_call(
        body,
        out_shape=(jax.ShapeDtypeStruct((N, DZ), jnp.float32),
                   jax.ShapeDtypeStruct((N, DZ), jnp.float32)),
    )(s1, aggp, inv, wmu, bmu, wvar, bvar)


def kernel(x, edge_index, W_self0, W_neigh0, b0, W_self1, W_neigh1, b1,
           W_mu, b_mu, W_var, b_var):
    n_edges = edge_index.shape[1]
    per_w = -(-n_edges // (NW * 4 * K)) * 4 * K
    pad = per_w * NW - n_edges
    pid = jnp.arange(pad, dtype=jnp.int32)
    srcs = jnp.concatenate([edge_index[0], pid % N]).reshape(NW, per_w // K, K)
    dsts = jnp.concatenate([edge_index[1], N + (pid % (N_PAD - N))]).reshape(
        NW, per_w // K, K)
    zeros_d = jnp.zeros((N_PAD, D), jnp.float32)
    zeros_flat = jnp.zeros((N_PAD // 128, 128), jnp.float32)

    degp = _edge_degree(dsts, zeros_flat)
    hn0, s0 = _tc_in(x, W_neigh0, W_self0, b0.reshape(1, D))
    agg0 = _edge_scatter(hn0, srcs, dsts, zeros_d)
    hn1, s1, inv = _tc_mid(s0, agg0, degp, W_neigh1, W_self1, b1.reshape(1, D))
    agg1 = _edge_scatter(hn1, srcs, dsts, zeros_d)
    return _tc_out(s1, agg1, inv, W_mu, b_mu.reshape(1, DZ),
                   W_var, b_var.reshape(1, DZ))


# speedup vs baseline: 1.1581x; 1.0194x over previous
import dataclasses

import jax
import jax.numpy as jnp
from jax import lax
from jax.experimental import pallas as pl
from jax.experimental.pallas import tpu as pltpu
from jax.experimental.pallas import tpu_sc as plsc

N = 10000
D = 128
DZ = 32
NC, NS = 2, 16
NW = NC * NS
K = 128
N_PAD = 10112
STRIPE = N_PAD // NS


def _sc_mesh():
    return plsc.VectorSubcoreMesh(
        core_axis_name="c", subcore_axis_name="s", num_cores=NC, num_subcores=NS
    )


def _edge_scatter(hn, srcs, dsts, zeros_d):
    n_chunks = srcs.shape[1]
    half = n_chunks // 2

    scratch = [
        pltpu.VMEM((half, K), jnp.int32),
        pltpu.VMEM((half, K), jnp.int32),
        pltpu.VMEM((K, D), jnp.float32),
        pltpu.VMEM((K, D), jnp.float32),
        pltpu.VMEM_SHARED((N_PAD, D), jnp.float32),
        pltpu.SemaphoreType.DMA,
        pltpu.SemaphoreType.DMA,
        pltpu.SemaphoreType.DMA,
    ]

    def body(hn_hbm, src_hbm, dst_hbm, zd_hbm, agg_out,
             src_v, dst_v, rows_a, rows_b, agg_sh, sem_a, sem_b, sem_z):
        cid = lax.axis_index("c")
        sid = lax.axis_index("s")
        wid = sid * NC + cid
        r0 = sid * STRIPE

        pltpu.async_copy(zd_hbm.at[pl.ds(r0, STRIPE)],
                         agg_sh.at[pl.ds(r0, STRIPE)], sem_z)
        pltpu.sync_copy(src_hbm.at[wid, pl.ds(0, half)], src_v)
        pltpu.sync_copy(dst_hbm.at[wid, pl.ds(0, half)], dst_v)
        pltpu.make_async_copy(zd_hbm.at[pl.ds(r0, STRIPE)],
                              agg_sh.at[pl.ds(r0, STRIPE)], sem_z).wait()
        plsc.subcore_barrier()

        def start(j, buf, sem):
            pltpu.async_copy(hn_hbm.at[src_v.at[j]], buf, sem)

        def wait(buf, sem):
            pltpu.make_async_copy(hn_hbm.at[src_v.at[0]], buf, sem).wait()

        def scat(j, buf):
            pltpu.sync_copy(buf, agg_sh.at[dst_v.at[j]], add=True)

        for h in range(2):
            if h:
                pltpu.sync_copy(src_hbm.at[wid, pl.ds(half, half)], src_v)
                pltpu.sync_copy(dst_hbm.at[wid, pl.ds(half, half)], dst_v)
            start(0, rows_a, sem_a)

            @pl.loop(0, half, step=2)
            def _(j):
                start(j + 1, rows_b, sem_b)
                wait(rows_a, sem_a)
                scat(j, rows_a)

                @pl.when(j + 2 < half)
                def _():
                    start(j + 2, rows_a, sem_a)

                wait(rows_b, sem_b)
                scat(j + 1, rows_b)

        plsc.subcore_barrier()
        pltpu.sync_copy(agg_sh.at[pl.ds(r0, STRIPE)],
                        agg_out.at[cid, pl.ds(r0, STRIPE)])

    run = pl.kernel(body,
                    out_type=jax.ShapeDtypeStruct((NC, N_PAD, D), jnp.float32),
                    mesh=_sc_mesh(), scratch_types=scratch)
    return run(hn, srcs, dsts, zeros_d)


def _edge_degree(dsts, zeros_flat):
    n_chunks = dsts.shape[1]

    scratch = [
        pltpu.VMEM((n_chunks, K), jnp.int32),
        pltpu.VMEM((N_PAD // 128, 128), jnp.float32),
    ]

    def body(dst_hbm, zf_hbm, hist_out, dst_v, hist_v):
        cid = lax.axis_index("c")
        sid = lax.axis_index("s")
        wid = sid * NC + cid
        pltpu.sync_copy(dst_hbm.at[wid], dst_v)
        pltpu.sync_copy(zf_hbm, hist_v)

        @pl.loop(0, n_chunks)
        def _(j):
            for t in range(K // 16):
                idx = dst_v[j, pl.ds(t * 16, 16)]
                cnt, last = plsc.scan_count(idx)
                plsc.addupdate_scatter(
                    hist_v, [idx >> 7, idx & 127],
                    cnt.astype(jnp.float32), mask=last)

        pltpu.sync_copy(hist_v, hist_out.at[wid])

    cp = pltpu.CompilerParams()
    if "needs_layout_passes" in pltpu.CompilerParams.__dataclass_fields__:
        cp = dataclasses.replace(cp, needs_layout_passes=False)
    run = pl.kernel(
        body,
        out_type=jax.ShapeDtypeStruct((NW, N_PAD // 128, 128), jnp.float32),
        mesh=_sc_mesh(), scratch_types=scratch, compiler_params=cp)
    return run(dsts, zeros_flat)


def _tc_in(x, wn, ws, b):
    def body(x_ref, wn_ref, ws_ref, b_ref, hn_ref, s_ref):
        h = jnp.log(x_ref[...] + 1.0)
        hn_ref[...] = jnp.dot(h, wn_ref[...], preferred_element_type=jnp.float32)
        s_ref[...] = jnp.dot(h, ws_ref[...],
                             preferred_element_type=jnp.float32) + b_ref[...]

    return pl.pallas_call(
        body,
        out_shape=(jax.ShapeDtypeStruct((N, D), jnp.float32),
                   jax.ShapeDtypeStruct((N, D), jnp.float32)),
    )(x, wn, ws, b)


def _finish_layer(s_ref, aggp_ref, inv_col):
    agg = aggp_ref[0, :N, :] + aggp_ref[1, :N, :]
    pre = jnp.maximum(s_ref[...] + agg * inv_col, 0.0)
    nrm = jnp.sqrt(jnp.sum(pre * pre, axis=1, keepdims=True))
    return pre / jnp.maximum(nrm, 1e-12)


def _deg_col(degp):
    def body(degp_ref, inv_ref):
        degsum = jnp.sum(degp_ref[...], axis=0)
        dt = jnp.transpose(degsum)
        col = jnp.concatenate([dt[:, c:c + 1] for c in range(N_PAD // 128)],
                              axis=0)
        inv_ref[...] = 1.0 / jnp.maximum(col[:N], 1.0)

    return pl.pallas_call(
        body, out_shape=jax.ShapeDtypeStruct((N, 1), jnp.float32))(degp)


def _tc_mid(s0, aggp, inv, wn, ws, b):
    def body(s0_ref, aggp_ref, inv_ref, wn_ref, ws_ref, b_ref, hn_ref, s_ref):
        h1 = _finish_layer(s0_ref, aggp_ref, inv_ref[...])
        hn_ref[...] = jnp.dot(h1, wn_ref[...], preferred_element_type=jnp.float32)
        s_ref[...] = jnp.dot(h1, ws_ref[...],
                             preferred_element_type=jnp.float32) + b_ref[...]

    return pl.pallas_call(
        body,
        out_shape=(jax.ShapeDtypeStruct((N, D), jnp.float32),
                   jax.ShapeDtypeStruct((N, D), jnp.float32)),
    )(s0, aggp, inv, wn, ws, b)


def _tc_out(s1, aggp, inv, wmu, bmu, wvar, bvar):
    def body(s1_ref, aggp_ref, inv_ref, wmu_ref, bmu_ref, wvar_ref, bvar_ref,
             zl_ref, zs_ref):
        h2 = _finish_layer(s1_ref, aggp_ref, inv_ref[...])
        zl_ref[...] = jnp.dot(h2, wmu_ref[...],
                              preferred_element_type=jnp.float32) + bmu_ref[...]
        zs_ref[...] = jnp.exp(jnp.dot(h2, wvar_ref[...],
                                      preferred_element_type=jnp.float32)
                              + bvar_ref[...]) + 1e-6

    return pl.pallas_call(
        body,
        out_shape=(jax.ShapeDtypeStruct((N, DZ), jnp.float32),
                   jax.ShapeDtypeStruct((N, DZ), jnp.float32)),
    )(s1, aggp, inv, wmu, bmu, wvar, bvar)


def kernel(x, edge_index, W_self0, W_neigh0, b0, W_self1, W_neigh1, b1,
           W_mu, b_mu, W_var, b_var):
    n_edges = edge_index.shape[1]
    per_w = -(-n_edges // (NW * 4 * K)) * 4 * K
    pad = per_w * NW - n_edges
    pid = jnp.arange(pad, dtype=jnp.int32)
    srcs = jnp.concatenate([edge_index[0], pid % N]).reshape(NW, per_w // K, K)
    dsts = jnp.concatenate([edge_index[1], N + (pid % (N_PAD - N))]).reshape(
        NW, per_w // K, K)
    zeros_d = jnp.zeros((N_PAD, D), jnp.float32)
    zeros_flat = jnp.zeros((N_PAD // 128, 128), jnp.float32)

    degp = _edge_degree(dsts, zeros_flat)
    hn0, s0 = _tc_in(x, W_neigh0, W_self0, b0.reshape(1, D))
    hn0 = lax.optimization_barrier((hn0, degp))[0]
    agg0 = _edge_scatter(hn0, srcs, dsts, zeros_d)
    inv = _deg_col(degp)
    hn1, s1 = _tc_mid(s0, agg0, inv, W_neigh1, W_self1, b1.reshape(1, D))
    agg1 = _edge_scatter(hn1, srcs, dsts, zeros_d)
    return _tc_out(s1, agg1, inv, W_mu, b_mu.reshape(1, DZ),
                   W_var, b_var.reshape(1, DZ))


# speedup vs baseline: 1.2017x; 1.0377x over previous
import dataclasses

import jax
import jax.numpy as jnp
from jax import lax
from jax.experimental import pallas as pl
from jax.experimental.pallas import tpu as pltpu
from jax.experimental.pallas import tpu_sc as plsc

N = 10000
D = 128
DZ = 32
NC, NS = 2, 16
NW = NC * NS
K = 128
N_PAD = 10112
STRIPE = N_PAD // NS


def _sc_mesh():
    return plsc.VectorSubcoreMesh(
        core_axis_name="c", subcore_axis_name="s", num_cores=NC, num_subcores=NS
    )


def _edge_scatter(hn, srcs, dsts, zeros_d):
    n_chunks = srcs.shape[1]
    half = n_chunks // 2

    scratch = [
        pltpu.VMEM((half, K), jnp.int32),
        pltpu.VMEM((half, K), jnp.int32),
        pltpu.VMEM((K, D), jnp.float32),
        pltpu.VMEM((K, D), jnp.float32),
        pltpu.VMEM_SHARED((N_PAD, D), jnp.float32),
        pltpu.SemaphoreType.DMA,
        pltpu.SemaphoreType.DMA,
        pltpu.SemaphoreType.DMA,
    ]

    def body(hn_hbm, src_hbm, dst_hbm, zd_hbm, agg_out,
             src_v, dst_v, rows_a, rows_b, agg_sh, sem_a, sem_b, sem_z):
        cid = lax.axis_index("c")
        sid = lax.axis_index("s")
        wid = sid * NC + cid
        r0 = sid * STRIPE

        pltpu.async_copy(zd_hbm.at[pl.ds(r0, STRIPE)],
                         agg_sh.at[pl.ds(r0, STRIPE)], sem_z)
        pltpu.sync_copy(src_hbm.at[wid, pl.ds(0, half)], src_v)
        pltpu.sync_copy(dst_hbm.at[wid, pl.ds(0, half)], dst_v)
        pltpu.make_async_copy(zd_hbm.at[pl.ds(r0, STRIPE)],
                              agg_sh.at[pl.ds(r0, STRIPE)], sem_z).wait()
        plsc.subcore_barrier()

        def start(j, buf, sem):
            pltpu.async_copy(hn_hbm.at[src_v.at[j]], buf, sem)

        def wait(buf, sem):
            pltpu.make_async_copy(hn_hbm.at[src_v.at[0]], buf, sem).wait()

        def scat(j, buf):
            pltpu.sync_copy(buf, agg_sh.at[dst_v.at[j]], add=True)

        for h in range(2):
            if h:
                pltpu.sync_copy(src_hbm.at[wid, pl.ds(half, half)], src_v)
                pltpu.sync_copy(dst_hbm.at[wid, pl.ds(half, half)], dst_v)
            start(0, rows_a, sem_a)

            @pl.loop(0, half, step=2)
            def _(j):
                start(j + 1, rows_b, sem_b)
                wait(rows_a, sem_a)
                scat(j, rows_a)

                @pl.when(j + 2 < half)
                def _():
                    start(j + 2, rows_a, sem_a)

                wait(rows_b, sem_b)
                scat(j + 1, rows_b)

        plsc.subcore_barrier()
        pltpu.sync_copy(agg_sh.at[pl.ds(r0, STRIPE)],
                        agg_out.at[cid, pl.ds(r0, STRIPE)])

    run = pl.kernel(body,
                    out_type=jax.ShapeDtypeStruct((NC, N_PAD, D), jnp.float32),
                    mesh=_sc_mesh(), scratch_types=scratch)
    return run(hn, srcs, dsts, zeros_d)


def _edge_degree(dsts, zeros_flat):
    n_chunks = dsts.shape[1]

    scratch = [
        pltpu.VMEM((n_chunks, K), jnp.int32),
        pltpu.VMEM((N_PAD // 128, 128), jnp.float32),
    ]

    def body(dst_hbm, zf_hbm, hist_out, dst_v, hist_v):
        cid = lax.axis_index("c")
        sid = lax.axis_index("s")
        wid = sid * NC + cid
        pltpu.sync_copy(dst_hbm.at[wid], dst_v)
        pltpu.sync_copy(zf_hbm, hist_v)

        @pl.loop(0, n_chunks)
        def _(j):
            for t in range(K // 16):
                idx = dst_v[j, pl.ds(t * 16, 16)]
                cnt, last = plsc.scan_count(idx)
                plsc.addupdate_scatter(
                    hist_v, [idx >> 7, idx & 127],
                    cnt.astype(jnp.float32), mask=last)

        pltpu.sync_copy(hist_v, hist_out.at[wid])

    cp = pltpu.CompilerParams()
    if "needs_layout_passes" in pltpu.CompilerParams.__dataclass_fields__:
        cp = dataclasses.replace(cp, needs_layout_passes=False)
    run = pl.kernel(
        body,
        out_type=jax.ShapeDtypeStruct((NW, N_PAD // 128, 128), jnp.float32),
        mesh=_sc_mesh(), scratch_types=scratch, compiler_params=cp)
    return run(dsts, zeros_flat)


def _tc_in(x, wn, ws, b):
    def body(x_ref, wn_ref, ws_ref, b_ref, hn_ref, s_ref):
        h = jnp.log(x_ref[...] + 1.0)
        hn_ref[...] = jnp.dot(h, wn_ref[...], preferred_element_type=jnp.float32)
        s_ref[...] = jnp.dot(h, ws_ref[...],
                             preferred_element_type=jnp.float32) + b_ref[...]

    return pl.pallas_call(
        body,
        out_shape=(jax.ShapeDtypeStruct((N, D), jnp.float32),
                   jax.ShapeDtypeStruct((N, D), jnp.float32)),
    )(x, wn, ws, b)


def _finish_layer(s_ref, aggp_ref, inv_col):
    agg = aggp_ref[0, :N, :] + aggp_ref[1, :N, :]
    pre = jnp.maximum(s_ref[...] + agg * inv_col, 0.0)
    nrm = jnp.sqrt(jnp.sum(pre * pre, axis=1, keepdims=True))
    return pre / jnp.maximum(nrm, 1e-12)


def _deg_col(degp):
    def body(degp_ref, inv_ref):
        degsum = jnp.sum(degp_ref[...], axis=0)
        dt = jnp.transpose(degsum)
        col = jnp.concatenate([dt[:, c:c + 1] for c in range(N_PAD // 128)],
                              axis=0)
        inv_ref[...] = 1.0 / jnp.maximum(col[:N], 1.0)

    return pl.pallas_call(
        body, out_shape=jax.ShapeDtypeStruct((N, 1), jnp.float32))(degp)


def _tc_mid(s0, aggp, inv, wn, ws, b):
    def body(s0_ref, aggp_ref, inv_ref, wn_ref, ws_ref, b_ref, hn_ref, s_ref):
        h1 = _finish_layer(s0_ref, aggp_ref, inv_ref[...])
        hn_ref[...] = jnp.dot(h1, wn_ref[...], preferred_element_type=jnp.float32)
        s_ref[...] = jnp.dot(h1, ws_ref[...],
                             preferred_element_type=jnp.float32) + b_ref[...]

    return pl.pallas_call(
        body,
        out_shape=(jax.ShapeDtypeStruct((N, D), jnp.float32),
                   jax.ShapeDtypeStruct((N, D), jnp.float32)),
    )(s0, aggp, inv, wn, ws, b)


def _tc_out(s1, aggp, inv, wmu, bmu, wvar, bvar):
    def body(s1_ref, aggp_ref, inv_ref, wmu_ref, bmu_ref, wvar_ref, bvar_ref,
             zl_ref, zs_ref):
        h2 = _finish_layer(s1_ref, aggp_ref, inv_ref[...])
        dn = (((0,), (1,)), ((), ()))
        zl_ref[...] = lax.dot_general(
            wmu_ref[...], h2, dn,
            preferred_element_type=jnp.float32) + bmu_ref[...]
        zs_ref[...] = jnp.exp(lax.dot_general(
            wvar_ref[...], h2, dn,
            preferred_element_type=jnp.float32) + bvar_ref[...]) + 1e-6

    zl_t, zs_t = pl.pallas_call(
        body,
        out_shape=(jax.ShapeDtypeStruct((DZ, N), jnp.float32),
                   jax.ShapeDtypeStruct((DZ, N), jnp.float32)),
    )(s1, aggp, inv, wmu, bmu, wvar, bvar)
    return jnp.transpose(zl_t), jnp.transpose(zs_t)


def kernel(x, edge_index, W_self0, W_neigh0, b0, W_self1, W_neigh1, b1,
           W_mu, b_mu, W_var, b_var):
    n_edges = edge_index.shape[1]
    per_w = -(-n_edges // (NW * 4 * K)) * 4 * K
    pad = per_w * NW - n_edges
    pid = jnp.arange(pad, dtype=jnp.int32)
    srcs = jnp.concatenate([edge_index[0], pid % N]).reshape(NW, per_w // K, K)
    dsts = jnp.concatenate([edge_index[1], N + (pid % (N_PAD - N))]).reshape(
        NW, per_w // K, K)
    zeros_d = jnp.zeros((N_PAD, D), jnp.float32)
    zeros_flat = jnp.zeros((N_PAD // 128, 128), jnp.float32)

    degp = _edge_degree(dsts, zeros_flat)
    hn0, s0 = _tc_in(x, W_neigh0, W_self0, b0.reshape(1, D))
    hn0 = lax.optimization_barrier((hn0, degp))[0]
    agg0 = _edge_scatter(hn0, srcs, dsts, zeros_d)
    inv = _deg_col(degp)
    hn1, s1 = _tc_mid(s0, agg0, inv, W_neigh1, W_self1, b1.reshape(1, D))
    agg1 = _edge_scatter(hn1, srcs, dsts, zeros_d)
    return _tc_out(s1, agg1, inv, W_mu, b_mu.reshape(DZ, 1),
                   W_var, b_var.reshape(DZ, 1))
